# baseline (speedup 1.0000x reference)
import math
import jax
import jax.numpy as jnp
from jax.experimental import pallas as pl
from jax.experimental.pallas import tpu as pltpu

_CAPACITY_FACTOR = 2.0
_MIN_CAPACITY = 4


def _capacity(s, e):
    c = math.floor(_CAPACITY_FACTOR * s / e)
    c += c % 2
    return max(c, _MIN_CAPACITY)


def _router_body(cap, x_ref, cw_ref, sm_ref, carry_ref):
    i = pl.program_id(0)
    t, e = x_ref.shape
    x = x_ref[...]

    xmax = jnp.max(x, axis=1, keepdims=True)
    denom = jnp.sum(jnp.exp(x - xmax), axis=1, keepdims=True)
    weight = 1.0 / denom

    colid = jax.lax.broadcasted_iota(jnp.int32, (t, e), 1)
    first = jnp.min(jnp.where(x == xmax, colid, e), axis=1, keepdims=True)
    oh = (colid == first).astype(jnp.float32)

    ri = jax.lax.broadcasted_iota(jnp.int32, (t, t), 0)
    ci = jax.lax.broadcasted_iota(jnp.int32, (t, t), 1)
    tril = (ri >= ci).astype(jnp.float32)
    cums = jnp.dot(tril, oh, preferred_element_type=jnp.float32)

    @pl.when(i == 0)
    def _init():
        carry_ref[...] = jnp.zeros_like(carry_ref)

    carry = carry_ref[0:1, :]
    rank = jnp.sum(oh * (cums + carry), axis=1, keepdims=True) - 1.0
    carry_ref[0:1, :] = carry + cums[t - 1 : t, :]

    keep = rank < cap
    target = jnp.where(keep, first * cap + rank.astype(jnp.int32), -1)

    cols = jax.lax.broadcasted_iota(jnp.int32, (t, e * cap), 1)
    hit = cols == target
    cw_ref[...] = jnp.where(hit, weight, 0.0)
    sm_ref[...] = hit


def kernel(inputs):
    s, e = inputs.shape
    cap = _capacity(s, e)
    blk = 256
    grid = s // blk

    body = lambda *refs: _router_body(cap, *refs)
    cw, sm = pl.pallas_call(
        body,
        grid=(grid,),
        in_specs=[pl.BlockSpec((blk, e), lambda i: (i, 0))],
        out_specs=[
            pl.BlockSpec((blk, e * cap), lambda i: (i, 0)),
            pl.BlockSpec((blk, e * cap), lambda i: (i, 0)),
        ],
        out_shape=[
            jax.ShapeDtypeStruct((s, e * cap), jnp.float32),
            jax.ShapeDtypeStruct((s, e * cap), jnp.bool_),
        ],
        scratch_shapes=[pltpu.VMEM((8, e), jnp.float32)],
    )(inputs.astype(jnp.float32))

    return cw.reshape(s, e, cap), sm.reshape(s, e, cap)


# speedup vs baseline: 2.0538x; 2.0538x over previous
import math
import jax
import jax.numpy as jnp
from jax.experimental import pallas as pl
from jax.experimental.pallas import tpu as pltpu

_CAPACITY_FACTOR = 2.0
_MIN_CAPACITY = 4


def _capacity(s, e):
    c = math.floor(_CAPACITY_FACTOR * s / e)
    c += c % 2
    return max(c, _MIN_CAPACITY)


def _router_body(cap, x_ref, cw_ref, sm_ref, carry_ref):
    i = pl.program_id(0)
    t, e = x_ref.shape
    x = x_ref[...]

    xmax = jnp.max(x, axis=1, keepdims=True)
    denom = jnp.sum(jnp.exp(x - xmax), axis=1, keepdims=True)
    weight = 1.0 / denom

    colid = jax.lax.broadcasted_iota(jnp.int32, (t, e), 1)
    first = jnp.min(jnp.where(x == xmax, colid, e), axis=1, keepdims=True)
    oh = (colid == first).astype(jnp.float32)

    ri = jax.lax.broadcasted_iota(jnp.int32, (t, t), 0)
    ci = jax.lax.broadcasted_iota(jnp.int32, (t, t), 1)
    tril = (ri >= ci).astype(jnp.float32)
    cums = jnp.dot(tril, oh, preferred_element_type=jnp.float32)

    @pl.when(i == 0)
    def _init():
        carry_ref[...] = jnp.zeros_like(carry_ref)

    carry = carry_ref[0:1, :]
    rank = jnp.sum(oh * (cums + carry), axis=1, keepdims=True) - 1.0
    carry_ref[0:1, :] = carry + cums[t - 1 : t, :]

    rankk = jnp.where(rank < cap, rank.astype(jnp.int32), -1).reshape(t, 1, 1)
    first3 = first.reshape(t, 1, 1)

    e_i = jax.lax.broadcasted_iota(jnp.int32, (t, e, 1), 1)
    c_i = jax.lax.broadcasted_iota(jnp.int32, (t, 1, cap), 2)
    hit = (e_i == first3) & (c_i == rankk)
    cw_ref[...] = jnp.where(hit, weight.reshape(t, 1, 1), 0.0)
    sm_ref[...] = hit


def kernel(inputs):
    s, e = inputs.shape
    cap = _capacity(s, e)
    blk = 256
    grid = s // blk

    body = lambda *refs: _router_body(cap, *refs)
    cw, sm = pl.pallas_call(
        body,
        grid=(grid,),
        in_specs=[pl.BlockSpec((blk, e), lambda i: (i, 0))],
        out_specs=[
            pl.BlockSpec((blk, e, cap), lambda i: (i, 0, 0)),
            pl.BlockSpec((blk, e, cap), lambda i: (i, 0, 0)),
        ],
        out_shape=[
            jax.ShapeDtypeStruct((s, e, cap), jnp.float32),
            jax.ShapeDtypeStruct((s, e, cap), jnp.bool_),
        ],
        scratch_shapes=[pltpu.VMEM((8, e), jnp.float32)],
    )(inputs.astype(jnp.float32))

    return cw, sm
